# baseline (speedup 1.0000x reference)
import functools

import jax
import jax.numpy as jnp
from jax.experimental import pallas as pl

K = 3


def _softmax_body(x_ref, o_ref):
    x = x_ref[...]
    m = jnp.max(x, axis=1, keepdims=True)
    e = jnp.exp(x - m)
    o_ref[...] = e / jnp.sum(e, axis=1, keepdims=True)


def _softmax(h):
    n, c = h.shape
    blk = 2048
    npad = ((n + blk - 1) // blk) * blk
    hp = jnp.pad(h, ((0, npad - n), (0, 0)))
    out = pl.pallas_call(
        _softmax_body,
        out_shape=jax.ShapeDtypeStruct((npad, c), h.dtype),
        grid=(npad // blk,),
        in_specs=[pl.BlockSpec((blk, c), lambda i: (i, 0))],
        out_specs=pl.BlockSpec((blk, c), lambda i: (i, 0)),
    )(hp)
    return out[:n]


def _gmm(x, ei, pseudo, p):
    g, mu, sigma, root, bias = p
    src, dst = ei[0], ei[1]
    out_c = g.shape[1] // K
    z = x @ g
    zj = jnp.take(z, src, axis=0).reshape(-1, K, out_c)
    d = pseudo[:, None, :] - mu[None, :, :]
    w = jnp.exp(-0.5 * jnp.sum(d * d / (sigma[None, :, :] ** 2 + 1e-16), axis=-1))
    msg = jnp.sum(zj * w[:, :, None], axis=1)
    n = x.shape[0]
    agg = jax.ops.segment_sum(msg, dst, num_segments=n)
    cnt = jax.ops.segment_sum(jnp.ones((msg.shape[0],), dtype=msg.dtype), dst, num_segments=n)
    agg = agg / jnp.maximum(cnt, 1.0)[:, None]
    return agg + x @ root + bias


def _hex_pool(x, hexa):
    g = jnp.take(x, hexa, axis=0)
    L = (x.shape[0] + 6) // 4
    vals = jnp.max(g, axis=1)[:L]
    a = jnp.argmax(g, axis=1)[:L]
    idx = jnp.take_along_axis(hexa[:L], a, axis=1)
    return vals, idx


def _hex_unpool(x, idx):
    L = x.shape[0] * 4 - 6
    C = x.shape[1]
    cols = jnp.broadcast_to(jnp.arange(C, dtype=idx.dtype), idx.shape)
    y = jnp.zeros((L, C), dtype=x.dtype)
    return y.at[idx, cols].set(x)


def kernel(x, edge_index, edges_coarse, pseudos, hexes, params):
    relu = jax.nn.relu
    p = params
    x0 = relu(_gmm(x, edge_index, pseudos[0], p[0]))
    x1, i1 = _hex_pool(x0, hexes[0])
    h = relu(_gmm(x1, edges_coarse[0], pseudos[1], p[1]))
    x2, i2 = _hex_pool(h, hexes[1])
    h = relu(_gmm(x2, edges_coarse[1], pseudos[2], p[2]))
    x3, i3 = _hex_pool(h, hexes[2])
    h = relu(_gmm(x3, edges_coarse[2], pseudos[3], p[3]))
    x4, i4 = _hex_pool(h, hexes[3])
    h = relu(_gmm(x4, edges_coarse[3], pseudos[4], p[4]))
    h = _hex_unpool(h, i4)
    h = jnp.concatenate([h, x3], axis=1)
    h = relu(_gmm(h, edges_coarse[2], pseudos[3], p[5]))
    h = _hex_unpool(h, i3)
    h = jnp.concatenate([h, x2], axis=1)
    h = relu(_gmm(h, edges_coarse[1], pseudos[2], p[6]))
    h = _hex_unpool(h, i2)
    h = jnp.concatenate([h, x1], axis=1)
    h = relu(_gmm(h, edges_coarse[0], pseudos[1], p[7]))
    h = _hex_unpool(h, i1)
    h = jnp.concatenate([h, x0], axis=1)
    h = _gmm(h, edge_index, pseudos[0], p[8])
    return _softmax(h)


# speedup vs baseline: 1.2684x; 1.2684x over previous
import dataclasses
import functools

import jax
import jax.numpy as jnp
from jax import lax
from jax.experimental import pallas as pl
from jax.experimental.pallas import tpu as pltpu
from jax.experimental.pallas import tpu_sc as plsc

K = 3
NTILES = 32
BLKN = 256

NLV = [40962, 10242, 2562, 642, 162]
ELV = [245760, 61440, 15360, 3840, 960]
SPECS = [(4, 32), (32, 64), (64, 128), (128, 256), (256, 256),
         (384, 128), (192, 64), (96, 32), (64, 21)]
CONV_LVL = [0, 1, 2, 3, 4, 3, 2, 1, 0]
CONV_CH = [256, 240, 96, 40, 32, 40, 240, 384, 256]


def _pad16(n):
    return ((n + 15) // 16) * 16


def _pad256(n):
    return ((n + 255) // 256) * 256


NP_ = [_pad256(n + 1) for n in NLV]
EP_ = [_pad256(e) for e in ELV]


def _mesh():
    return plsc.VectorSubcoreMesh(core_axis_name="c", subcore_axis_name="s")


def _sc_compiler_params():
    cp = pltpu.CompilerParams()
    if "needs_layout_passes" in pltpu.CompilerParams.__dataclass_fields__:
        cp = dataclasses.replace(cp, needs_layout_passes=False)
    if "use_tc_tiling_on_sc" in pltpu.CompilerParams.__dataclass_fields__:
        cp = dataclasses.replace(cp, use_tc_tiling_on_sc=False)
    return cp


def _full16(v):
    return jnp.full((16,), v, dtype=jnp.int32)


def _prep_body(x_ref, g_ref, r_ref, b_ref, z_ref, xr_ref):
    x = x_ref[...]
    z_ref[...] = jnp.dot(x, g_ref[...], preferred_element_type=jnp.float32)
    xr_ref[...] = jnp.dot(x, r_ref[...], preferred_element_type=jnp.float32) + b_ref[...]


def _prep(x, gp, rootp, biasp):
    npad, in_c = x.shape
    koutp = gp.shape[1]
    outp = rootp.shape[1]
    return pl.pallas_call(
        _prep_body,
        out_shape=(jax.ShapeDtypeStruct((npad, koutp), jnp.float32),
                   jax.ShapeDtypeStruct((npad, outp), jnp.float32)),
        grid=(npad // BLKN,),
        in_specs=[pl.BlockSpec((BLKN, in_c), lambda i: (i, 0)),
                  pl.BlockSpec((in_c, koutp), lambda i: (0, 0)),
                  pl.BlockSpec((in_c, outp), lambda i: (0, 0)),
                  pl.BlockSpec((1, outp), lambda i: (0, 0))],
        out_specs=(pl.BlockSpec((BLKN, koutp), lambda i: (i, 0)),
                   pl.BlockSpec((BLKN, outp), lambda i: (i, 0))),
    )(x, gp, rootp, biasp)


def _prep2_body(yt_ref, xs_ref, g1_ref, g2_ref, r1_ref, r2_ref, b_ref,
                z_ref, xr_ref):
    yt = yt_ref[...]
    xs = xs_ref[...]
    dn = (((0,), (0,)), ((), ()))
    z_ref[...] = (lax.dot_general(yt, g1_ref[...], dn, preferred_element_type=jnp.float32)
                  + jnp.dot(xs, g2_ref[...], preferred_element_type=jnp.float32))
    xr_ref[...] = (lax.dot_general(yt, r1_ref[...], dn, preferred_element_type=jnp.float32)
                   + jnp.dot(xs, r2_ref[...], preferred_element_type=jnp.float32)
                   + b_ref[...])


def _prep2(yt, xs, gp, rootp, biasp):
    c1 = yt.shape[0]
    npad, c2 = xs.shape
    koutp = gp.shape[1]
    outp = rootp.shape[1]
    g1, g2 = gp[:c1], gp[c1:]
    r1, r2 = rootp[:c1], rootp[c1:]
    return pl.pallas_call(
        _prep2_body,
        out_shape=(jax.ShapeDtypeStruct((npad, koutp), jnp.float32),
                   jax.ShapeDtypeStruct((npad, outp), jnp.float32)),
        grid=(npad // BLKN,),
        in_specs=[pl.BlockSpec((c1, BLKN), lambda i: (0, i)),
                  pl.BlockSpec((BLKN, c2), lambda i: (i, 0)),
                  pl.BlockSpec((c1, koutp), lambda i: (0, 0)),
                  pl.BlockSpec((c2, koutp), lambda i: (0, 0)),
                  pl.BlockSpec((c1, outp), lambda i: (0, 0)),
                  pl.BlockSpec((c2, outp), lambda i: (0, 0)),
                  pl.BlockSpec((1, outp), lambda i: (0, 0))],
        out_specs=(pl.BlockSpec((BLKN, koutp), lambda i: (i, 0)),
                   pl.BlockSpec((BLKN, outp), lambda i: (i, 0))),
    )(yt, xs, g1, g2, r1, r2, biasp)


def _wgt_body(ps_ref, ms_ref, w_ref):
    u = ps_ref[:, 0:1]
    v = ps_ref[:, 1:2]
    cols = []
    for k in range(K):
        m0 = ms_ref[k, 0]
        m1 = ms_ref[k, 1]
        s0 = ms_ref[k + K, 0]
        s1 = ms_ref[k + K, 1]
        e = -0.5 * ((u - m0) ** 2 / (s0 * s0 + 1e-16)
                    + (v - m1) ** 2 / (s1 * s1 + 1e-16))
        cols.append(jnp.exp(e))
    blke = u.shape[0]
    cols.append(jnp.zeros((blke, 8 - K), jnp.float32))
    w_ref[...] = jnp.concatenate(cols, axis=1)


def _wgt(ps, mu, sigma):
    ep = ps.shape[0]
    blke = min(ep, 3840)
    assert ep % blke == 0, (ep, blke)
    ms = jnp.concatenate([mu, sigma], axis=0)
    return pl.pallas_call(
        _wgt_body,
        out_shape=jax.ShapeDtypeStruct((ep, 8), jnp.float32),
        grid=(ep // blke,),
        in_specs=[pl.BlockSpec((blke, 2), lambda i: (i, 0)),
                  pl.BlockSpec(memory_space=pltpu.SMEM)],
        out_specs=pl.BlockSpec((blke, 8), lambda i: (i, 0)),
    )(ps, ms)


def _combine_body(a0_ref, a1_ref, c0_ref, c1_ref, xr_ref, o_ref,
                  *, outp, act, ncls):
    a = a0_ref[...] + a1_ref[...]
    cnt = jnp.maximum(c0_ref[:, 0:1] + c1_ref[:, 0:1], 1.0)
    h = a / cnt + xr_ref[...]
    if act == "relu":
        o_ref[...] = jnp.maximum(h, 0.0)
    else:
        h = h[:, :ncls]
        m = jnp.max(h, axis=1, keepdims=True)
        e = jnp.exp(h - m)
        o_ref[...] = e / jnp.sum(e, axis=1, keepdims=True)


def _combine(aggc, cntc, xr, outp, act="relu", ncls=0):
    npad = xr.shape[0]
    ocols = outp if act == "relu" else ncls
    body = functools.partial(_combine_body, outp=outp, act=act, ncls=ncls)
    return pl.pallas_call(
        body,
        out_shape=jax.ShapeDtypeStruct((npad, ocols), jnp.float32),
        grid=(npad // BLKN,),
        in_specs=[pl.BlockSpec((BLKN, outp), lambda i: (i, 0)),
                  pl.BlockSpec((BLKN, outp), lambda i: (i, 0)),
                  pl.BlockSpec((BLKN, 16), lambda i: (i, 0)),
                  pl.BlockSpec((BLKN, 16), lambda i: (i, 0)),
                  pl.BlockSpec((BLKN, outp), lambda i: (i, 0))],
        out_specs=pl.BlockSpec((BLKN, ocols), lambda i: (i, 0)),
    )(aggc[0], aggc[1], cntc[0], cntc[1], xr)


def _edge_agg_body(z_hbm, src_hbm, dst_hbm, w_hbm, out_hbm,
                   src_v, dst_v, w_v, rows_v, msg_v, acc_sh,
                   *, np_rows, koutp, outp, ept, ch, ep):
    c_idx = lax.axis_index("c")
    s_idx = lax.axis_index("s")
    tile = s_idx * 2 + c_idx

    zeros16 = jnp.zeros((16,), jnp.float32)

    @pl.loop(0, ch)
    def _(r):
        for c in range(outp // 16):
            msg_v[r, pl.ds(c * 16, 16)] = zeros16

    rps = np_rows // 16
    nfull = rps // ch
    tail = rps % ch
    base_z = s_idx * rps
    for i in range(nfull):
        pltpu.sync_copy(msg_v, acc_sh.at[pl.ds(base_z + i * ch, ch)])
    if tail:
        pltpu.sync_copy(msg_v.at[pl.ds(0, tail)],
                        acc_sh.at[pl.ds(base_z + nfull * ch, tail)])
    plsc.subcore_barrier()

    nchunks = ept // ch
    for ci in range(nchunks):
        base = tile * ept + ci * ch
        pltpu.sync_copy(src_hbm.at[pl.ds(base, ch)], src_v)
        pltpu.sync_copy(dst_hbm.at[pl.ds(base, ch)], dst_v)
        pltpu.sync_copy(w_hbm.at[pl.ds(base, ch), :], w_v)
        pltpu.sync_copy(z_hbm.at[src_v], rows_v)

        @pl.loop(0, ch)
        def _(e):
            e16 = _full16(e)
            w0 = plsc.load_gather(w_v, [e16, _full16(0)])
            w1 = plsc.load_gather(w_v, [e16, _full16(1)])
            w2 = plsc.load_gather(w_v, [e16, _full16(2)])
            for c in range(outp // 16):
                v = (w0 * rows_v[e, pl.ds(c * 16, 16)]
                     + w1 * rows_v[e, pl.ds(outp + c * 16, 16)]
                     + w2 * rows_v[e, pl.ds(2 * outp + c * 16, 16)])
                msg_v[e, pl.ds(c * 16, 16)] = v

        pltpu.sync_copy(msg_v, acc_sh.at[dst_v], add=True)

    plsc.subcore_barrier()
    for i in range(nfull):
        pltpu.sync_copy(acc_sh.at[pl.ds(base_z + i * ch, ch)],
                        out_hbm.at[c_idx, pl.ds(base_z + i * ch, ch), :])
    if tail:
        pltpu.sync_copy(acc_sh.at[pl.ds(base_z + nfull * ch, tail)],
                        out_hbm.at[c_idx, pl.ds(base_z + nfull * ch, tail), :])


def _edge_agg(z, srcp, dstp, wflat, np_rows, koutp, outp, ch):
    ep = srcp.shape[0]
    ept = ep // NTILES
    body = functools.partial(
        _edge_agg_body, np_rows=np_rows, koutp=koutp, outp=outp,
        ept=ept, ch=ch, ep=ep)
    k = pl.kernel(
        body,
        out_type=jax.ShapeDtypeStruct((2, np_rows, outp), jnp.float32),
        mesh=_mesh(),
        scratch_types=[
            pltpu.VMEM((ch,), jnp.int32),
            pltpu.VMEM((ch,), jnp.int32),
            pltpu.VMEM((ch, 8), jnp.float32),
            pltpu.VMEM((ch, koutp), jnp.float32),
            pltpu.VMEM((ch, outp), jnp.float32),
            pltpu.VMEM_SHARED((np_rows, outp), jnp.float32),
        ],
        compiler_params=_sc_compiler_params(),
    )
    return k(z, srcp, dstp, wflat)


def _count_body(dst_hbm, out_hbm, dst_v, ones_v, acc_sh, *, np_rows, ept, ch):
    c_idx = lax.axis_index("c")
    s_idx = lax.axis_index("s")
    tile = s_idx * 2 + c_idx

    zeros16 = jnp.zeros((16,), jnp.float32)
    ones16 = jnp.ones((16,), jnp.float32)

    @pl.loop(0, ch)
    def _(r):
        ones_v[r, pl.ds(0, 16)] = zeros16

    rps = np_rows // 16
    nfull = rps // ch
    tail = rps % ch
    base_z = s_idx * rps
    for i in range(nfull):
        pltpu.sync_copy(ones_v, acc_sh.at[pl.ds(base_z + i * ch, ch)])
    if tail:
        pltpu.sync_copy(ones_v.at[pl.ds(0, tail)],
                        acc_sh.at[pl.ds(base_z + nfull * ch, tail)])

    @pl.loop(0, ch)
    def _(r):
        ones_v[r, pl.ds(0, 16)] = ones16

    plsc.subcore_barrier()

    for ci in range(ept // ch):
        base = tile * ept + ci * ch
        pltpu.sync_copy(dst_hbm.at[pl.ds(base, ch)], dst_v)
        pltpu.sync_copy(ones_v, acc_sh.at[dst_v], add=True)

    plsc.subcore_barrier()
    for i in range(nfull):
        pltpu.sync_copy(acc_sh.at[pl.ds(base_z + i * ch, ch)],
                        out_hbm.at[c_idx, pl.ds(base_z + i * ch, ch), :])
    if tail:
        pltpu.sync_copy(acc_sh.at[pl.ds(base_z + nfull * ch, tail)],
                        out_hbm.at[c_idx, pl.ds(base_z + nfull * ch, tail), :])


def _count(dstp, np_rows, ch):
    ep = dstp.shape[0]
    ept = ep // NTILES
    body = functools.partial(_count_body, np_rows=np_rows, ept=ept, ch=ch)
    k = pl.kernel(
        body,
        out_type=jax.ShapeDtypeStruct((2, np_rows, 16), jnp.float32),
        mesh=_mesh(),
        scratch_types=[
            pltpu.VMEM((ch,), jnp.int32),
            pltpu.VMEM((ch, 16), jnp.float32),
            pltpu.VMEM_SHARED((np_rows, 16), jnp.float32),
        ],
        compiler_params=_sc_compiler_params(),
    )
    return k(dstp)


def _pad_rows(x, npad):
    return jnp.pad(x, ((0, npad - x.shape[0]), (0, 0)))


def _prep_conv_params(p, in_c, out_c):
    g, mu, sigma, root, bias = p
    outp = _pad16(out_c)
    gp = g.reshape(in_c, K, out_c)
    gp = jnp.pad(gp, ((0, 0), (0, 0), (0, outp - out_c))).reshape(in_c, K * outp)
    rootp = jnp.pad(root, ((0, 0), (0, outp - out_c)))
    biasp = jnp.pad(bias, (0, outp - out_c)).reshape(1, outp)
    return gp, rootp, biasp, mu, sigma, outp


def _gmm_sc(xin, lvl, conv_i, edge_data, cntc, params, act="relu", ncls=0,
            yt=None):
    srcp, dstp, ps = edge_data
    in_c, out_c = SPECS[conv_i]
    gp, rootp, biasp, mu, sigma, outp = _prep_conv_params(params[conv_i], in_c, out_c)
    if yt is None:
        z, xr = _prep(xin, gp, rootp, biasp)
    else:
        z, xr = _prep2(yt, xin, gp, rootp, biasp)
    w8 = _wgt(ps, mu, sigma)
    aggc = _edge_agg(z, srcp, dstp, w8, NP_[lvl], K * outp, outp,
                     CONV_CH[conv_i])
    return _combine(aggc, cntc, xr, outp, act=act, ncls=ncls)


def _hex_pool_jax(x, hexa, n_real, np_out):
    g = jnp.take(x[:n_real], hexa, axis=0)
    L = (n_real + 6) // 4
    vals = jnp.max(g, axis=1)[:L]
    a = jnp.argmax(g, axis=1)[:L]
    idx = jnp.take_along_axis(hexa[:L], a, axis=1)
    return _pad_rows(vals, np_out), idx


def _hex_unpool_jax(x, idx, nf, nfp):
    C = x.shape[1]
    cols = jnp.broadcast_to(jnp.arange(C, dtype=idx.dtype), idx.shape)
    y = jnp.zeros((nf, C), dtype=x.dtype)
    y = y.at[idx, cols].set(x)
    return jnp.pad(y.T, ((0, 0), (0, nfp - nf)))


def kernel(x, edge_index, edges_coarse, pseudos, hexes, params):
    edges = [edge_index] + list(edges_coarse)
    edata = []
    cnts = []
    cnt_ch = [512, 480, 480, 120, 32]
    for l in range(5):
        e, ep_, n = ELV[l], EP_[l], NLV[l]
        src = jnp.pad(edges[l][0], (0, ep_ - e))
        dst = jnp.pad(edges[l][1], (0, ep_ - e), constant_values=n)
        ps = jnp.pad(pseudos[l], ((0, ep_ - e), (0, 0)))
        edata.append((src, dst, ps))
        cnts.append(_count(dst, NP_[l], cnt_ch[l]))

    xp = _pad_rows(x, NP_[0])

    x0 = _gmm_sc(xp, 0, 0, edata[0], cnts[0], params)
    x1, i1 = _hex_pool_jax(x0, hexes[0], NLV[0], NP_[1])
    h = _gmm_sc(x1, 1, 1, edata[1], cnts[1], params)
    x2, i2 = _hex_pool_jax(h, hexes[1], NLV[1], NP_[2])
    h = _gmm_sc(x2, 2, 2, edata[2], cnts[2], params)
    x3, i3 = _hex_pool_jax(h, hexes[2], NLV[2], NP_[3])
    h = _gmm_sc(x3, 3, 3, edata[3], cnts[3], params)
    x4, i4 = _hex_pool_jax(h, hexes[3], NLV[3], NP_[4])
    h = _gmm_sc(x4, 4, 4, edata[4], cnts[4], params)

    yt = _hex_unpool_jax(h[:NLV[4]], i4, NLV[3], NP_[3])
    h = _gmm_sc(x3, 3, 5, edata[3], cnts[3], params, yt=yt)
    yt = _hex_unpool_jax(h[:NLV[3]], i3, NLV[2], NP_[2])
    h = _gmm_sc(x2, 2, 6, edata[2], cnts[2], params, yt=yt)
    yt = _hex_unpool_jax(h[:NLV[2]], i2, NLV[1], NP_[1])
    h = _gmm_sc(x1, 1, 7, edata[1], cnts[1], params, yt=yt)
    yt = _hex_unpool_jax(h[:NLV[1]], i1, NLV[0], NP_[0])
    out = _gmm_sc(x0, 0, 8, edata[0], cnts[0], params, act="softmax", ncls=21,
                  yt=yt)
    return out[:NLV[0]]


# speedup vs baseline: 4.5858x; 3.6155x over previous
import dataclasses
import functools

import jax
import jax.numpy as jnp
from jax import lax
from jax.experimental import pallas as pl
from jax.experimental.pallas import tpu as pltpu
from jax.experimental.pallas import tpu_sc as plsc

K = 3
NTILES = 32
BLKN = 256

NLV = [40962, 10242, 2562, 642, 162]
ELV = [245760, 61440, 15360, 3840, 960]
SPECS = [(4, 32), (32, 64), (64, 128), (128, 256), (256, 256),
         (384, 128), (192, 64), (96, 32), (64, 21)]
CONV_LVL = [0, 1, 2, 3, 4, 3, 2, 1, 0]
CONV_CH = [256, 240, 96, 40, 32, 40, 240, 384, 256]


def _pad16(n):
    return ((n + 15) // 16) * 16


def _pad256(n):
    return ((n + 255) // 256) * 256


NP_ = [_pad256(n + 1) for n in NLV]
EP_ = [_pad256(e) for e in ELV]


def _mesh():
    return plsc.VectorSubcoreMesh(core_axis_name="c", subcore_axis_name="s")


def _sc_compiler_params():
    cp = pltpu.CompilerParams()
    if "needs_layout_passes" in pltpu.CompilerParams.__dataclass_fields__:
        cp = dataclasses.replace(cp, needs_layout_passes=False)
    if "use_tc_tiling_on_sc" in pltpu.CompilerParams.__dataclass_fields__:
        cp = dataclasses.replace(cp, use_tc_tiling_on_sc=False)
    return cp


def _full16(v):
    return jnp.full((16,), v, dtype=jnp.int32)


def _prep_body(x_ref, g_ref, r_ref, b_ref, z_ref, xr_ref):
    x = x_ref[...]
    z_ref[...] = jnp.dot(x, g_ref[...], preferred_element_type=jnp.float32)
    xr_ref[...] = jnp.dot(x, r_ref[...], preferred_element_type=jnp.float32) + b_ref[...]


def _prep(x, gp, rootp, biasp):
    npad, in_c = x.shape
    koutp = gp.shape[1]
    outp = rootp.shape[1]
    return pl.pallas_call(
        _prep_body,
        out_shape=(jax.ShapeDtypeStruct((npad, koutp), jnp.float32),
                   jax.ShapeDtypeStruct((npad, outp), jnp.float32)),
        grid=(npad // BLKN,),
        in_specs=[pl.BlockSpec((BLKN, in_c), lambda i: (i, 0)),
                  pl.BlockSpec((in_c, koutp), lambda i: (0, 0)),
                  pl.BlockSpec((in_c, outp), lambda i: (0, 0)),
                  pl.BlockSpec((1, outp), lambda i: (0, 0))],
        out_specs=(pl.BlockSpec((BLKN, koutp), lambda i: (i, 0)),
                   pl.BlockSpec((BLKN, outp), lambda i: (i, 0))),
    )(x, gp, rootp, biasp)


def _prep2_body(yt_ref, xs_ref, g1_ref, g2_ref, r1_ref, r2_ref, b_ref,
                z_ref, xr_ref):
    yt = yt_ref[...]
    xs = xs_ref[...]
    dn = (((0,), (0,)), ((), ()))
    z_ref[...] = (lax.dot_general(yt, g1_ref[...], dn, preferred_element_type=jnp.float32)
                  + jnp.dot(xs, g2_ref[...], preferred_element_type=jnp.float32))
    xr_ref[...] = (lax.dot_general(yt, r1_ref[...], dn, preferred_element_type=jnp.float32)
                   + jnp.dot(xs, r2_ref[...], preferred_element_type=jnp.float32)
                   + b_ref[...])


def _prep2(yt, xs, gp, rootp, biasp):
    c1 = yt.shape[0]
    npad, c2 = xs.shape
    koutp = gp.shape[1]
    outp = rootp.shape[1]
    g1, g2 = gp[:c1], gp[c1:]
    r1, r2 = rootp[:c1], rootp[c1:]
    return pl.pallas_call(
        _prep2_body,
        out_shape=(jax.ShapeDtypeStruct((npad, koutp), jnp.float32),
                   jax.ShapeDtypeStruct((npad, outp), jnp.float32)),
        grid=(npad // BLKN,),
        in_specs=[pl.BlockSpec((c1, BLKN), lambda i: (0, i)),
                  pl.BlockSpec((BLKN, c2), lambda i: (i, 0)),
                  pl.BlockSpec((c1, koutp), lambda i: (0, 0)),
                  pl.BlockSpec((c2, koutp), lambda i: (0, 0)),
                  pl.BlockSpec((c1, outp), lambda i: (0, 0)),
                  pl.BlockSpec((c2, outp), lambda i: (0, 0)),
                  pl.BlockSpec((1, outp), lambda i: (0, 0))],
        out_specs=(pl.BlockSpec((BLKN, koutp), lambda i: (i, 0)),
                   pl.BlockSpec((BLKN, outp), lambda i: (i, 0))),
    )(yt, xs, g1, g2, r1, r2, biasp)


def _wgt_body(ps_ref, ms_ref, w_ref):
    u = ps_ref[:, 0:1]
    v = ps_ref[:, 1:2]
    cols = []
    for k in range(K):
        m0 = ms_ref[k, 0]
        m1 = ms_ref[k, 1]
        s0 = ms_ref[k + K, 0]
        s1 = ms_ref[k + K, 1]
        e = -0.5 * ((u - m0) ** 2 / (s0 * s0 + 1e-16)
                    + (v - m1) ** 2 / (s1 * s1 + 1e-16))
        cols.append(jnp.exp(e))
    blke = u.shape[0]
    cols.append(jnp.zeros((blke, 8 - K), jnp.float32))
    w_ref[...] = jnp.concatenate(cols, axis=1)


def _wgt(ps, mu, sigma):
    ep = ps.shape[0]
    blke = min(ep, 3840)
    assert ep % blke == 0, (ep, blke)
    ms = jnp.concatenate([mu, sigma], axis=0)
    return pl.pallas_call(
        _wgt_body,
        out_shape=jax.ShapeDtypeStruct((ep, 8), jnp.float32),
        grid=(ep // blke,),
        in_specs=[pl.BlockSpec((blke, 2), lambda i: (i, 0)),
                  pl.BlockSpec(memory_space=pltpu.SMEM)],
        out_specs=pl.BlockSpec((blke, 8), lambda i: (i, 0)),
    )(ps, ms)


def _combine_body(a0_ref, a1_ref, c0_ref, c1_ref, xr_ref, o_ref,
                  *, outp, act, ncls):
    a = a0_ref[...] + a1_ref[...]
    cnt = jnp.maximum(c0_ref[:, 0:1] + c1_ref[:, 0:1], 1.0)
    h = a / cnt + xr_ref[...]
    if act == "relu":
        o_ref[...] = jnp.maximum(h, 0.0)
    else:
        h = h[:, :ncls]
        m = jnp.max(h, axis=1, keepdims=True)
        e = jnp.exp(h - m)
        o_ref[...] = e / jnp.sum(e, axis=1, keepdims=True)


def _combine(aggc, cntc, xr, outp, act="relu", ncls=0):
    npad = xr.shape[0]
    ocols = outp if act == "relu" else ncls
    body = functools.partial(_combine_body, outp=outp, act=act, ncls=ncls)
    return pl.pallas_call(
        body,
        out_shape=jax.ShapeDtypeStruct((npad, ocols), jnp.float32),
        grid=(npad // BLKN,),
        in_specs=[pl.BlockSpec((BLKN, outp), lambda i: (i, 0)),
                  pl.BlockSpec((BLKN, outp), lambda i: (i, 0)),
                  pl.BlockSpec((BLKN, 16), lambda i: (i, 0)),
                  pl.BlockSpec((BLKN, 16), lambda i: (i, 0)),
                  pl.BlockSpec((BLKN, outp), lambda i: (i, 0))],
        out_specs=pl.BlockSpec((BLKN, ocols), lambda i: (i, 0)),
    )(aggc[0], aggc[1], cntc[0], cntc[1], xr)


def _edge_agg_body(z_hbm, src_hbm, dst_hbm, w_hbm, out_hbm,
                   src_v, dst_v, w_v, rows_v, msg_v, acc_sh,
                   *, np_rows, koutp, outp, ept, ch, ep):
    c_idx = lax.axis_index("c")
    s_idx = lax.axis_index("s")
    tile = s_idx * 2 + c_idx

    zeros16 = jnp.zeros((16,), jnp.float32)

    @pl.loop(0, ch)
    def _(r):
        for c in range(outp // 16):
            msg_v[r, pl.ds(c * 16, 16)] = zeros16

    rps = np_rows // 16
    nfull = rps // ch
    tail = rps % ch
    base_z = s_idx * rps
    for i in range(nfull):
        pltpu.sync_copy(msg_v, acc_sh.at[pl.ds(base_z + i * ch, ch)])
    if tail:
        pltpu.sync_copy(msg_v.at[pl.ds(0, tail)],
                        acc_sh.at[pl.ds(base_z + nfull * ch, tail)])
    plsc.subcore_barrier()

    nchunks = ept // ch
    for ci in range(nchunks):
        base = tile * ept + ci * ch
        pltpu.sync_copy(src_hbm.at[pl.ds(base, ch)], src_v)
        pltpu.sync_copy(dst_hbm.at[pl.ds(base, ch)], dst_v)
        pltpu.sync_copy(w_hbm.at[pl.ds(base, ch), :], w_v)
        pltpu.sync_copy(z_hbm.at[src_v], rows_v)

        @pl.loop(0, ch)
        def _(e):
            e16 = _full16(e)
            w0 = plsc.load_gather(w_v, [e16, _full16(0)])
            w1 = plsc.load_gather(w_v, [e16, _full16(1)])
            w2 = plsc.load_gather(w_v, [e16, _full16(2)])
            for c in range(outp // 16):
                v = (w0 * rows_v[e, pl.ds(c * 16, 16)]
                     + w1 * rows_v[e, pl.ds(outp + c * 16, 16)]
                     + w2 * rows_v[e, pl.ds(2 * outp + c * 16, 16)])
                msg_v[e, pl.ds(c * 16, 16)] = v

        pltpu.sync_copy(msg_v, acc_sh.at[dst_v], add=True)

    plsc.subcore_barrier()
    for i in range(nfull):
        pltpu.sync_copy(acc_sh.at[pl.ds(base_z + i * ch, ch)],
                        out_hbm.at[c_idx, pl.ds(base_z + i * ch, ch), :])
    if tail:
        pltpu.sync_copy(acc_sh.at[pl.ds(base_z + nfull * ch, tail)],
                        out_hbm.at[c_idx, pl.ds(base_z + nfull * ch, tail), :])


def _edge_agg(z, srcp, dstp, wflat, np_rows, koutp, outp, ch):
    ep = srcp.shape[0]
    ept = ep // NTILES
    body = functools.partial(
        _edge_agg_body, np_rows=np_rows, koutp=koutp, outp=outp,
        ept=ept, ch=ch, ep=ep)
    k = pl.kernel(
        body,
        out_type=jax.ShapeDtypeStruct((2, np_rows, outp), jnp.float32),
        mesh=_mesh(),
        scratch_types=[
            pltpu.VMEM((ch,), jnp.int32),
            pltpu.VMEM((ch,), jnp.int32),
            pltpu.VMEM((ch, 8), jnp.float32),
            pltpu.VMEM((ch, koutp), jnp.float32),
            pltpu.VMEM((ch, outp), jnp.float32),
            pltpu.VMEM_SHARED((np_rows, outp), jnp.float32),
        ],
        compiler_params=_sc_compiler_params(),
    )
    return k(z, srcp, dstp, wflat)


def _count_body(dst_hbm, out_hbm, dst_v, ones_v, acc_sh, *, np_rows, ept, ch):
    c_idx = lax.axis_index("c")
    s_idx = lax.axis_index("s")
    tile = s_idx * 2 + c_idx

    zeros16 = jnp.zeros((16,), jnp.float32)
    ones16 = jnp.ones((16,), jnp.float32)

    @pl.loop(0, ch)
    def _(r):
        ones_v[r, pl.ds(0, 16)] = zeros16

    rps = np_rows // 16
    nfull = rps // ch
    tail = rps % ch
    base_z = s_idx * rps
    for i in range(nfull):
        pltpu.sync_copy(ones_v, acc_sh.at[pl.ds(base_z + i * ch, ch)])
    if tail:
        pltpu.sync_copy(ones_v.at[pl.ds(0, tail)],
                        acc_sh.at[pl.ds(base_z + nfull * ch, tail)])

    @pl.loop(0, ch)
    def _(r):
        ones_v[r, pl.ds(0, 16)] = ones16

    plsc.subcore_barrier()

    for ci in range(ept // ch):
        base = tile * ept + ci * ch
        pltpu.sync_copy(dst_hbm.at[pl.ds(base, ch)], dst_v)
        pltpu.sync_copy(ones_v, acc_sh.at[dst_v], add=True)

    plsc.subcore_barrier()
    for i in range(nfull):
        pltpu.sync_copy(acc_sh.at[pl.ds(base_z + i * ch, ch)],
                        out_hbm.at[c_idx, pl.ds(base_z + i * ch, ch), :])
    if tail:
        pltpu.sync_copy(acc_sh.at[pl.ds(base_z + nfull * ch, tail)],
                        out_hbm.at[c_idx, pl.ds(base_z + nfull * ch, tail), :])


def _count(dstp, np_rows, ch):
    ep = dstp.shape[0]
    ept = ep // NTILES
    body = functools.partial(_count_body, np_rows=np_rows, ept=ept, ch=ch)
    k = pl.kernel(
        body,
        out_type=jax.ShapeDtypeStruct((2, np_rows, 16), jnp.float32),
        mesh=_mesh(),
        scratch_types=[
            pltpu.VMEM((ch,), jnp.int32),
            pltpu.VMEM((ch, 16), jnp.float32),
            pltpu.VMEM_SHARED((np_rows, 16), jnp.float32),
        ],
        compiler_params=_sc_compiler_params(),
    )
    return k(dstp)


def _pool_body(x_hbm, hexf_hbm, vals_hbm, idx_hbm,
               h0, h1, h2, h3, h4, h5, h6, rows_v, vals_v, idx_v,
               *, rpt, c, npo):
    c_idx = lax.axis_index("c")
    s_idx = lax.axis_index("s")
    tile = s_idx * 2 + c_idx
    base = tile * rpt
    hv = [h0, h1, h2, h3, h4, h5, h6]
    for j in range(7):
        pltpu.sync_copy(hexf_hbm.at[pl.ds(j * npo + base, rpt)], hv[j])
        pltpu.sync_copy(x_hbm.at[hv[j]], rows_v.at[j])

    @pl.loop(0, rpt)
    def _(r):
        r16 = _full16(r)
        hidx = [plsc.load_gather(hv[j], [r16]) for j in range(7)]
        for cc in range(c // 16):
            best = rows_v[0, r, pl.ds(cc * 16, 16)]
            bidx = hidx[0]
            for j in range(1, 7):
                cand = rows_v[j, r, pl.ds(cc * 16, 16)]
                m = cand > best
                best = jnp.where(m, cand, best)
                bidx = jnp.where(m, hidx[j], bidx)
            vals_v[r, pl.ds(cc * 16, 16)] = best
            idx_v[r, pl.ds(cc * 16, 16)] = bidx

    pltpu.sync_copy(vals_v, vals_hbm.at[pl.ds(base, rpt), :])
    pltpu.sync_copy(idx_v, idx_hbm.at[pl.ds(base, rpt), :])


def _pool_sc(x, hexf, npo, c):
    rpt = npo // NTILES
    body = functools.partial(_pool_body, rpt=rpt, c=c, npo=npo)
    k = pl.kernel(
        body,
        out_type=(jax.ShapeDtypeStruct((npo, c), jnp.float32),
                  jax.ShapeDtypeStruct((npo, c), jnp.int32)),
        mesh=_mesh(),
        scratch_types=[pltpu.VMEM((rpt,), jnp.int32) for _ in range(7)] + [
            pltpu.VMEM((7, rpt, c), jnp.float32),
            pltpu.VMEM((rpt, c), jnp.float32),
            pltpu.VMEM((rpt, c), jnp.int32),
        ],
        compiler_params=_sc_compiler_params(),
    )
    return k(x, hexf)


def _unpool_body(xt_hbm, it_hbm, yt_hbm, x_v, i_v, stripe_v,
                 *, cpt, lp, nfp):
    c_idx = lax.axis_index("c")
    s_idx = lax.axis_index("s")
    tile = s_idx * 2 + c_idx
    c0 = tile * cpt
    pltpu.sync_copy(xt_hbm.at[pl.ds(c0, cpt), :], x_v)
    pltpu.sync_copy(it_hbm.at[pl.ds(c0, cpt), :], i_v)

    zeros16 = jnp.zeros((16,), jnp.float32)
    for c in range(cpt):
        @pl.loop(0, nfp, step=16)
        def _(r):
            stripe_v[c, pl.ds(r, 16)] = zeros16

    for c in range(cpt):
        c16 = _full16(c)

        @pl.loop(0, lp, step=16)
        def _(l):
            xv = x_v[c, pl.ds(l, 16)]
            iv = i_v[c, pl.ds(l, 16)]
            plsc.store_scatter(stripe_v, [c16, iv], xv)

    pltpu.sync_copy(stripe_v, yt_hbm.at[pl.ds(c0, cpt), :])


def _unpool_sc(xt, it, nfp):
    c, lp = xt.shape
    cpt = c // NTILES
    body = functools.partial(_unpool_body, cpt=cpt, lp=lp, nfp=nfp)
    k = pl.kernel(
        body,
        out_type=jax.ShapeDtypeStruct((c, nfp), jnp.float32),
        mesh=_mesh(),
        scratch_types=[
            pltpu.VMEM((cpt, lp), jnp.float32),
            pltpu.VMEM((cpt, lp), jnp.int32),
            pltpu.VMEM((cpt, nfp), jnp.float32),
        ],
        compiler_params=_sc_compiler_params(),
    )
    return k(xt, it)


def _pad_rows(x, npad):
    return jnp.pad(x, ((0, npad - x.shape[0]), (0, 0)))


def _prep_conv_params(p, in_c, out_c):
    g, mu, sigma, root, bias = p
    outp = _pad16(out_c)
    gp = g.reshape(in_c, K, out_c)
    gp = jnp.pad(gp, ((0, 0), (0, 0), (0, outp - out_c))).reshape(in_c, K * outp)
    rootp = jnp.pad(root, ((0, 0), (0, outp - out_c)))
    biasp = jnp.pad(bias, (0, outp - out_c)).reshape(1, outp)
    return gp, rootp, biasp, mu, sigma, outp


def _gmm_sc(xin, lvl, conv_i, edge_data, cntc, params, act="relu", ncls=0,
            yt=None):
    srcp, dstp, ps = edge_data
    in_c, out_c = SPECS[conv_i]
    gp, rootp, biasp, mu, sigma, outp = _prep_conv_params(params[conv_i], in_c, out_c)
    if yt is None:
        z, xr = _prep(xin, gp, rootp, biasp)
    else:
        z, xr = _prep2(yt, xin, gp, rootp, biasp)
    w8 = _wgt(ps, mu, sigma)
    aggc = _edge_agg(z, srcp, dstp, w8, NP_[lvl], K * outp, outp,
                     CONV_CH[conv_i])
    return _combine(aggc, cntc, xr, outp, act=act, ncls=ncls)


def _hex_pool_jax(x, hexa, n_real, np_out):
    g = jnp.take(x[:n_real], hexa, axis=0)
    L = (n_real + 6) // 4
    vals = jnp.max(g, axis=1)[:L]
    a = jnp.argmax(g, axis=1)[:L]
    idx = jnp.take_along_axis(hexa[:L], a, axis=1)
    return _pad_rows(vals, np_out), idx


def _hex_unpool_jax(x, idx, nf, nfp):
    C = x.shape[1]
    cols = jnp.broadcast_to(jnp.arange(C, dtype=idx.dtype), idx.shape)
    y = jnp.zeros((nf, C), dtype=x.dtype)
    y = y.at[idx, cols].set(x)
    return jnp.pad(y.T, ((0, 0), (0, nfp - nf)))


def kernel(x, edge_index, edges_coarse, pseudos, hexes, params):
    edges = [edge_index] + list(edges_coarse)
    edata = []
    cnts = []
    cnt_ch = [512, 480, 480, 120, 32]
    for l in range(5):
        e, ep_, n = ELV[l], EP_[l], NLV[l]
        src = jnp.pad(edges[l][0], (0, ep_ - e))
        dst = jnp.pad(edges[l][1], (0, ep_ - e), constant_values=n)
        ps = jnp.pad(pseudos[l], ((0, ep_ - e), (0, 0)))
        edata.append((src, dst, ps))
        cnts.append(_count(dst, NP_[l], cnt_ch[l]))

    xp = _pad_rows(x, NP_[0])
    hexf = [jnp.reshape(hexes[l][:NP_[l + 1]].T, (7 * NP_[l + 1],))
            for l in range(4)]
    lpc = [_pad16(n) for n in NLV]

    def unpool_in(h, idx, lvl_c):
        L, lp = NLV[lvl_c], lpc[lvl_c]
        xt = jnp.pad(h[:L].T, ((0, 0), (0, lp - L)))
        it = jnp.pad(idx[:L].T, ((0, 0), (0, lp - L)),
                     constant_values=NLV[lvl_c - 1])
        return xt, it

    x0 = _gmm_sc(xp, 0, 0, edata[0], cnts[0], params)
    x1, i1 = _pool_sc(x0, hexf[0], NP_[1], 32)
    h = _gmm_sc(x1, 1, 1, edata[1], cnts[1], params)
    x2, i2 = _pool_sc(h, hexf[1], NP_[2], 64)
    h = _gmm_sc(x2, 2, 2, edata[2], cnts[2], params)
    x3, i3 = _pool_sc(h, hexf[2], NP_[3], 128)
    h = _gmm_sc(x3, 3, 3, edata[3], cnts[3], params)
    x4, i4 = _pool_sc(h, hexf[3], NP_[4], 256)
    h = _gmm_sc(x4, 4, 4, edata[4], cnts[4], params)

    yt = _unpool_sc(*unpool_in(h, i4, 4), NP_[3])
    h = _gmm_sc(x3, 3, 5, edata[3], cnts[3], params, yt=yt)
    yt = _unpool_sc(*unpool_in(h, i3, 3), NP_[2])
    h = _gmm_sc(x2, 2, 6, edata[2], cnts[2], params, yt=yt)
    yt = _unpool_sc(*unpool_in(h, i2, 2), NP_[1])
    h = _gmm_sc(x1, 1, 7, edata[1], cnts[1], params, yt=yt)
    yt = _unpool_sc(*unpool_in(h, i1, 1), NP_[0])
    out = _gmm_sc(x0, 0, 8, edata[0], cnts[0], params, act="softmax", ncls=21,
                  yt=yt)
    return out[:NLV[0]]


# speedup vs baseline: 4.9455x; 1.0784x over previous
import dataclasses
import functools

import jax
import jax.numpy as jnp
from jax import lax
from jax.experimental import pallas as pl
from jax.experimental.pallas import tpu as pltpu
from jax.experimental.pallas import tpu_sc as plsc

K = 3
NTILES = 32
BLKN = 256

NLV = [40962, 10242, 2562, 642, 162]
ELV = [245760, 61440, 15360, 3840, 960]
SPECS = [(4, 32), (32, 64), (64, 128), (128, 256), (256, 256),
         (384, 128), (192, 64), (96, 32), (64, 21)]
CONV_LVL = [0, 1, 2, 3, 4, 3, 2, 1, 0]
CONV_CH = [192, 120, 96, 40, 32, 40, 96, 384, 192]


def _pad16(n):
    return ((n + 15) // 16) * 16


def _pad256(n):
    return ((n + 255) // 256) * 256


NP_ = [_pad256(n + 1) for n in NLV]
EP_ = [_pad256(e) for e in ELV]


def _mesh():
    return plsc.VectorSubcoreMesh(core_axis_name="c", subcore_axis_name="s")


def _sc_compiler_params():
    cp = pltpu.CompilerParams()
    if "needs_layout_passes" in pltpu.CompilerParams.__dataclass_fields__:
        cp = dataclasses.replace(cp, needs_layout_passes=False)
    if "use_tc_tiling_on_sc" in pltpu.CompilerParams.__dataclass_fields__:
        cp = dataclasses.replace(cp, use_tc_tiling_on_sc=False)
    return cp


def _full16(v):
    return jnp.full((16,), v, dtype=jnp.int32)


def _prep_body(x_ref, g_ref, r_ref, b_ref, z_ref, xr_ref):
    x = x_ref[...]
    z_ref[...] = jnp.dot(x, g_ref[...], preferred_element_type=jnp.float32)
    xr_ref[...] = jnp.dot(x, r_ref[...], preferred_element_type=jnp.float32) + b_ref[...]


def _prep(x, gp, rootp, biasp):
    npad, in_c = x.shape
    koutp = gp.shape[1]
    outp = rootp.shape[1]
    return pl.pallas_call(
        _prep_body,
        out_shape=(jax.ShapeDtypeStruct((npad, koutp), jnp.float32),
                   jax.ShapeDtypeStruct((npad, outp), jnp.float32)),
        grid=(npad // BLKN,),
        in_specs=[pl.BlockSpec((BLKN, in_c), lambda i: (i, 0)),
                  pl.BlockSpec((in_c, koutp), lambda i: (0, 0)),
                  pl.BlockSpec((in_c, outp), lambda i: (0, 0)),
                  pl.BlockSpec((1, outp), lambda i: (0, 0))],
        out_specs=(pl.BlockSpec((BLKN, koutp), lambda i: (i, 0)),
                   pl.BlockSpec((BLKN, outp), lambda i: (i, 0))),
    )(x, gp, rootp, biasp)


def _prep2_body(yt_ref, xs_ref, g1_ref, g2_ref, r1_ref, r2_ref, b_ref,
                z_ref, xr_ref):
    yt = yt_ref[...]
    xs = xs_ref[...]
    dn = (((0,), (0,)), ((), ()))
    z_ref[...] = (lax.dot_general(yt, g1_ref[...], dn, preferred_element_type=jnp.float32)
                  + jnp.dot(xs, g2_ref[...], preferred_element_type=jnp.float32))
    xr_ref[...] = (lax.dot_general(yt, r1_ref[...], dn, preferred_element_type=jnp.float32)
                   + jnp.dot(xs, r2_ref[...], preferred_element_type=jnp.float32)
                   + b_ref[...])


def _prep2(yt, xs, gp, rootp, biasp):
    c1 = yt.shape[0]
    npad, c2 = xs.shape
    koutp = gp.shape[1]
    outp = rootp.shape[1]
    g1, g2 = gp[:c1], gp[c1:]
    r1, r2 = rootp[:c1], rootp[c1:]
    return pl.pallas_call(
        _prep2_body,
        out_shape=(jax.ShapeDtypeStruct((npad, koutp), jnp.float32),
                   jax.ShapeDtypeStruct((npad, outp), jnp.float32)),
        grid=(npad // BLKN,),
        in_specs=[pl.BlockSpec((c1, BLKN), lambda i: (0, i)),
                  pl.BlockSpec((BLKN, c2), lambda i: (i, 0)),
                  pl.BlockSpec((c1, koutp), lambda i: (0, 0)),
                  pl.BlockSpec((c2, koutp), lambda i: (0, 0)),
                  pl.BlockSpec((c1, outp), lambda i: (0, 0)),
                  pl.BlockSpec((c2, outp), lambda i: (0, 0)),
                  pl.BlockSpec((1, outp), lambda i: (0, 0))],
        out_specs=(pl.BlockSpec((BLKN, koutp), lambda i: (i, 0)),
                   pl.BlockSpec((BLKN, outp), lambda i: (i, 0))),
    )(yt, xs, g1, g2, r1, r2, biasp)


def _wgt_body(ps_ref, ms_ref, w_ref):
    u = ps_ref[:, 0:1]
    v = ps_ref[:, 1:2]
    cols = []
    for k in range(K):
        m0 = ms_ref[k, 0]
        m1 = ms_ref[k, 1]
        s0 = ms_ref[k + K, 0]
        s1 = ms_ref[k + K, 1]
        e = -0.5 * ((u - m0) ** 2 / (s0 * s0 + 1e-16)
                    + (v - m1) ** 2 / (s1 * s1 + 1e-16))
        cols.append(jnp.exp(e))
    blke = u.shape[0]
    cols.append(jnp.zeros((blke, 8 - K), jnp.float32))
    w_ref[...] = jnp.concatenate(cols, axis=1)


def _wgt(ps, mu, sigma):
    ep = ps.shape[0]
    blke = min(ep, 3840)
    assert ep % blke == 0, (ep, blke)
    ms = jnp.concatenate([mu, sigma], axis=0)
    return pl.pallas_call(
        _wgt_body,
        out_shape=jax.ShapeDtypeStruct((ep, 8), jnp.float32),
        grid=(ep // blke,),
        in_specs=[pl.BlockSpec((blke, 2), lambda i: (i, 0)),
                  pl.BlockSpec(memory_space=pltpu.SMEM)],
        out_specs=pl.BlockSpec((blke, 8), lambda i: (i, 0)),
    )(ps, ms)


def _combine_body(a0_ref, a1_ref, c0_ref, c1_ref, xr_ref, o_ref,
                  *, outp, act, ncls):
    a = a0_ref[...] + a1_ref[...]
    cnt = jnp.maximum(c0_ref[:, 0:1] + c1_ref[:, 0:1], 1.0)
    h = a / cnt + xr_ref[...]
    if act == "relu":
        o_ref[...] = jnp.maximum(h, 0.0)
    else:
        h = h[:, :ncls]
        m = jnp.max(h, axis=1, keepdims=True)
        e = jnp.exp(h - m)
        o_ref[...] = e / jnp.sum(e, axis=1, keepdims=True)


def _combine(aggc, cntc, xr, outp, act="relu", ncls=0):
    npad = xr.shape[0]
    ocols = outp if act == "relu" else ncls
    body = functools.partial(_combine_body, outp=outp, act=act, ncls=ncls)
    return pl.pallas_call(
        body,
        out_shape=jax.ShapeDtypeStruct((npad, ocols), jnp.float32),
        grid=(npad // BLKN,),
        in_specs=[pl.BlockSpec((BLKN, outp), lambda i: (i, 0)),
                  pl.BlockSpec((BLKN, outp), lambda i: (i, 0)),
                  pl.BlockSpec((BLKN, 16), lambda i: (i, 0)),
                  pl.BlockSpec((BLKN, 16), lambda i: (i, 0)),
                  pl.BlockSpec((BLKN, outp), lambda i: (i, 0))],
        out_specs=pl.BlockSpec((BLKN, ocols), lambda i: (i, 0)),
    )(aggc[0], aggc[1], cntc[0], cntc[1], xr)


def _edge_agg_body(z_hbm, src_hbm, dst_hbm, w_hbm, out_hbm,
                   src0, src1, dst0, dst1, w0v, w1v, rows0, rows1,
                   msg_v, acc_sh, semi0, semi1, semg0, semg1,
                   *, np_rows, koutp, outp, ept, ch, ep):
    c_idx = lax.axis_index("c")
    s_idx = lax.axis_index("s")
    tile = s_idx * 2 + c_idx

    zeros16 = jnp.zeros((16,), jnp.float32)

    @pl.loop(0, ch)
    def _(r):
        for c in range(outp // 16):
            msg_v[r, pl.ds(c * 16, 16)] = zeros16

    rps = np_rows // 16
    nfull = rps // ch
    tail = rps % ch
    base_z = s_idx * rps
    for i in range(nfull):
        pltpu.sync_copy(msg_v, acc_sh.at[pl.ds(base_z + i * ch, ch)])
    if tail:
        pltpu.sync_copy(msg_v.at[pl.ds(0, tail)],
                        acc_sh.at[pl.ds(base_z + nfull * ch, tail)])
    plsc.subcore_barrier()

    nchunks = ept // ch
    srcs = [src0, src1]
    dsts = [dst0, dst1]
    ws = [w0v, w1v]
    rows = [rows0, rows1]
    sem_i = [semi0, semi1]
    sem_g = [semg0, semg1]

    def start_idx(ci):
        b = ci % 2
        base = tile * ept + ci * ch
        return (pltpu.async_copy(src_hbm.at[pl.ds(base, ch)], srcs[b], sem_i[b]),
                pltpu.async_copy(dst_hbm.at[pl.ds(base, ch)], dsts[b], sem_i[b]),
                pltpu.async_copy(w_hbm.at[pl.ds(base, ch), :], ws[b], sem_i[b]))

    def start_gather(ci):
        b = ci % 2
        return pltpu.async_copy(z_hbm.at[srcs[b]], rows[b], sem_g[b])

    idx_h = start_idx(0)
    for h in idx_h:
        h.wait()
    g_h = start_gather(0)
    idx_next = start_idx(1) if nchunks > 1 else None

    for ci in range(nchunks):
        b = ci % 2
        g_h.wait()
        if ci + 1 < nchunks:
            for h in idx_next:
                h.wait()
            g_h = start_gather(ci + 1)

        rv = rows[b]
        wv = ws[b]

        @pl.loop(0, ch)
        def _(e):
            e16 = _full16(e)
            w0 = plsc.load_gather(wv, [e16, _full16(0)])
            w1 = plsc.load_gather(wv, [e16, _full16(1)])
            w2 = plsc.load_gather(wv, [e16, _full16(2)])
            for c in range(outp // 16):
                v = (w0 * rv[e, pl.ds(c * 16, 16)]
                     + w1 * rv[e, pl.ds(outp + c * 16, 16)]
                     + w2 * rv[e, pl.ds(2 * outp + c * 16, 16)])
                msg_v[e, pl.ds(c * 16, 16)] = v

        pltpu.sync_copy(msg_v, acc_sh.at[dsts[b]], add=True)
        if ci + 2 < nchunks:
            idx_next = start_idx(ci + 2)

    plsc.subcore_barrier()
    for i in range(nfull):
        pltpu.sync_copy(acc_sh.at[pl.ds(base_z + i * ch, ch)],
                        out_hbm.at[c_idx, pl.ds(base_z + i * ch, ch), :])
    if tail:
        pltpu.sync_copy(acc_sh.at[pl.ds(base_z + nfull * ch, tail)],
                        out_hbm.at[c_idx, pl.ds(base_z + nfull * ch, tail), :])


def _edge_agg(z, srcp, dstp, wflat, np_rows, koutp, outp, ch):
    ep = srcp.shape[0]
    ept = ep // NTILES
    body = functools.partial(
        _edge_agg_body, np_rows=np_rows, koutp=koutp, outp=outp,
        ept=ept, ch=ch, ep=ep)
    k = pl.kernel(
        body,
        out_type=jax.ShapeDtypeStruct((2, np_rows, outp), jnp.float32),
        mesh=_mesh(),
        scratch_types=[
            pltpu.VMEM((ch,), jnp.int32),
            pltpu.VMEM((ch,), jnp.int32),
            pltpu.VMEM((ch,), jnp.int32),
            pltpu.VMEM((ch,), jnp.int32),
            pltpu.VMEM((ch, 8), jnp.float32),
            pltpu.VMEM((ch, 8), jnp.float32),
            pltpu.VMEM((ch, koutp), jnp.float32),
            pltpu.VMEM((ch, koutp), jnp.float32),
            pltpu.VMEM((ch, outp), jnp.float32),
            pltpu.VMEM_SHARED((np_rows, outp), jnp.float32),
            pltpu.SemaphoreType.DMA,
            pltpu.SemaphoreType.DMA,
            pltpu.SemaphoreType.DMA,
            pltpu.SemaphoreType.DMA,
        ],
        compiler_params=_sc_compiler_params(),
    )
    return k(z, srcp, dstp, wflat)


def _count_body(dst_hbm, out_hbm, dst_v, ones_v, acc_sh, *, np_rows, ept, ch):
    c_idx = lax.axis_index("c")
    s_idx = lax.axis_index("s")
    tile = s_idx * 2 + c_idx

    zeros16 = jnp.zeros((16,), jnp.float32)
    ones16 = jnp.ones((16,), jnp.float32)

    @pl.loop(0, ch)
    def _(r):
        ones_v[r, pl.ds(0, 16)] = zeros16

    rps = np_rows // 16
    nfull = rps // ch
    tail = rps % ch
    base_z = s_idx * rps
    for i in range(nfull):
        pltpu.sync_copy(ones_v, acc_sh.at[pl.ds(base_z + i * ch, ch)])
    if tail:
        pltpu.sync_copy(ones_v.at[pl.ds(0, tail)],
                        acc_sh.at[pl.ds(base_z + nfull * ch, tail)])

    @pl.loop(0, ch)
    def _(r):
        ones_v[r, pl.ds(0, 16)] = ones16

    plsc.subcore_barrier()

    for ci in range(ept // ch):
        base = tile * ept + ci * ch
        pltpu.sync_copy(dst_hbm.at[pl.ds(base, ch)], dst_v)
        pltpu.sync_copy(ones_v, acc_sh.at[dst_v], add=True)

    plsc.subcore_barrier()
    for i in range(nfull):
        pltpu.sync_copy(acc_sh.at[pl.ds(base_z + i * ch, ch)],
                        out_hbm.at[c_idx, pl.ds(base_z + i * ch, ch), :])
    if tail:
        pltpu.sync_copy(acc_sh.at[pl.ds(base_z + nfull * ch, tail)],
                        out_hbm.at[c_idx, pl.ds(base_z + nfull * ch, tail), :])


def _count(dstp, np_rows, ch):
    ep = dstp.shape[0]
    ept = ep // NTILES
    body = functools.partial(_count_body, np_rows=np_rows, ept=ept, ch=ch)
    k = pl.kernel(
        body,
        out_type=jax.ShapeDtypeStruct((2, np_rows, 16), jnp.float32),
        mesh=_mesh(),
        scratch_types=[
            pltpu.VMEM((ch,), jnp.int32),
            pltpu.VMEM((ch, 16), jnp.float32),
            pltpu.VMEM_SHARED((np_rows, 16), jnp.float32),
        ],
        compiler_params=_sc_compiler_params(),
    )
    return k(dstp)


def _pool_body(x_hbm, hexf_hbm, vals_hbm, idx_hbm,
               h0, h1, h2, h3, h4, h5, h6, rows_v, vals_v, idx_v,
               *, rpt, c, npo):
    c_idx = lax.axis_index("c")
    s_idx = lax.axis_index("s")
    tile = s_idx * 2 + c_idx
    base = tile * rpt
    hv = [h0, h1, h2, h3, h4, h5, h6]
    for j in range(7):
        pltpu.sync_copy(hexf_hbm.at[pl.ds(j * npo + base, rpt)], hv[j])
        pltpu.sync_copy(x_hbm.at[hv[j]], rows_v.at[j])

    @pl.loop(0, rpt)
    def _(r):
        r16 = _full16(r)
        hidx = [plsc.load_gather(hv[j], [r16]) for j in range(7)]
        for cc in range(c // 16):
            best = rows_v[0, r, pl.ds(cc * 16, 16)]
            bidx = hidx[0]
            for j in range(1, 7):
                cand = rows_v[j, r, pl.ds(cc * 16, 16)]
                m = cand > best
                best = jnp.where(m, cand, best)
                bidx = jnp.where(m, hidx[j], bidx)
            vals_v[r, pl.ds(cc * 16, 16)] = best
            idx_v[r, pl.ds(cc * 16, 16)] = bidx

    pltpu.sync_copy(vals_v, vals_hbm.at[pl.ds(base, rpt), :])
    pltpu.sync_copy(idx_v, idx_hbm.at[pl.ds(base, rpt), :])


def _pool_sc(x, hexf, npo, c):
    rpt = npo // NTILES
    body = functools.partial(_pool_body, rpt=rpt, c=c, npo=npo)
    k = pl.kernel(
        body,
        out_type=(jax.ShapeDtypeStruct((npo, c), jnp.float32),
                  jax.ShapeDtypeStruct((npo, c), jnp.int32)),
        mesh=_mesh(),
        scratch_types=[pltpu.VMEM((rpt,), jnp.int32) for _ in range(7)] + [
            pltpu.VMEM((7, rpt, c), jnp.float32),
            pltpu.VMEM((rpt, c), jnp.float32),
            pltpu.VMEM((rpt, c), jnp.int32),
        ],
        compiler_params=_sc_compiler_params(),
    )
    return k(x, hexf)


def _unpool_body(xt_hbm, it_hbm, yt_hbm, x_v, i_v, stripe_v,
                 *, cpt, lp, nfp):
    c_idx = lax.axis_index("c")
    s_idx = lax.axis_index("s")
    tile = s_idx * 2 + c_idx
    c0 = tile * cpt
    pltpu.sync_copy(xt_hbm.at[pl.ds(c0, cpt), :], x_v)
    pltpu.sync_copy(it_hbm.at[pl.ds(c0, cpt), :], i_v)

    zeros16 = jnp.zeros((16,), jnp.float32)
    for c in range(cpt):
        @pl.loop(0, nfp, step=16)
        def _(r):
            stripe_v[c, pl.ds(r, 16)] = zeros16

    for c in range(cpt):
        c16 = _full16(c)

        @pl.loop(0, lp, step=16)
        def _(l):
            xv = x_v[c, pl.ds(l, 16)]
            iv = i_v[c, pl.ds(l, 16)]
            plsc.store_scatter(stripe_v, [c16, iv], xv)

    pltpu.sync_copy(stripe_v, yt_hbm.at[pl.ds(c0, cpt), :])


def _unpool_sc(xt, it, nfp):
    c, lp = xt.shape
    cpt = c // NTILES
    body = functools.partial(_unpool_body, cpt=cpt, lp=lp, nfp=nfp)
    k = pl.kernel(
        body,
        out_type=jax.ShapeDtypeStruct((c, nfp), jnp.float32),
        mesh=_mesh(),
        scratch_types=[
            pltpu.VMEM((cpt, lp), jnp.float32),
            pltpu.VMEM((cpt, lp), jnp.int32),
            pltpu.VMEM((cpt, nfp), jnp.float32),
        ],
        compiler_params=_sc_compiler_params(),
    )
    return k(xt, it)


def _pad_rows(x, npad):
    return jnp.pad(x, ((0, npad - x.shape[0]), (0, 0)))


def _prep_conv_params(p, in_c, out_c):
    g, mu, sigma, root, bias = p
    outp = _pad16(out_c)
    gp = g.reshape(in_c, K, out_c)
    gp = jnp.pad(gp, ((0, 0), (0, 0), (0, outp - out_c))).reshape(in_c, K * outp)
    rootp = jnp.pad(root, ((0, 0), (0, outp - out_c)))
    biasp = jnp.pad(bias, (0, outp - out_c)).reshape(1, outp)
    return gp, rootp, biasp, mu, sigma, outp


def _gmm_sc(xin, lvl, conv_i, edge_data, cntc, params, act="relu", ncls=0,
            yt=None):
    srcp, dstp, ps = edge_data
    in_c, out_c = SPECS[conv_i]
    gp, rootp, biasp, mu, sigma, outp = _prep_conv_params(params[conv_i], in_c, out_c)
    if yt is None:
        z, xr = _prep(xin, gp, rootp, biasp)
    else:
        z, xr = _prep2(yt, xin, gp, rootp, biasp)
    w8 = _wgt(ps, mu, sigma)
    aggc = _edge_agg(z, srcp, dstp, w8, NP_[lvl], K * outp, outp,
                     CONV_CH[conv_i])
    return _combine(aggc, cntc, xr, outp, act=act, ncls=ncls)


def _hex_pool_jax(x, hexa, n_real, np_out):
    g = jnp.take(x[:n_real], hexa, axis=0)
    L = (n_real + 6) // 4
    vals = jnp.max(g, axis=1)[:L]
    a = jnp.argmax(g, axis=1)[:L]
    idx = jnp.take_along_axis(hexa[:L], a, axis=1)
    return _pad_rows(vals, np_out), idx


def _hex_unpool_jax(x, idx, nf, nfp):
    C = x.shape[1]
    cols = jnp.broadcast_to(jnp.arange(C, dtype=idx.dtype), idx.shape)
    y = jnp.zeros((nf, C), dtype=x.dtype)
    y = y.at[idx, cols].set(x)
    return jnp.pad(y.T, ((0, 0), (0, nfp - nf)))


def kernel(x, edge_index, edges_coarse, pseudos, hexes, params):
    edges = [edge_index] + list(edges_coarse)
    edata = []
    cnts = []
    cnt_ch = [512, 480, 480, 120, 32]
    for l in range(5):
        e, ep_, n = ELV[l], EP_[l], NLV[l]
        src = jnp.pad(edges[l][0], (0, ep_ - e))
        dst = jnp.pad(edges[l][1], (0, ep_ - e), constant_values=n)
        ps = jnp.pad(pseudos[l], ((0, ep_ - e), (0, 0)))
        edata.append((src, dst, ps))
        cnts.append(_count(dst, NP_[l], cnt_ch[l]))

    xp = _pad_rows(x, NP_[0])
    hexf = [jnp.reshape(hexes[l][:NP_[l + 1]].T, (7 * NP_[l + 1],))
            for l in range(4)]
    lpc = [_pad16(n) for n in NLV]

    def unpool_in(h, idx, lvl_c):
        L, lp = NLV[lvl_c], lpc[lvl_c]
        xt = jnp.pad(h[:L].T, ((0, 0), (0, lp - L)))
        it = jnp.pad(idx[:L].T, ((0, 0), (0, lp - L)),
                     constant_values=NLV[lvl_c - 1])
        return xt, it

    x0 = _gmm_sc(xp, 0, 0, edata[0], cnts[0], params)
    x1, i1 = _pool_sc(x0, hexf[0], NP_[1], 32)
    h = _gmm_sc(x1, 1, 1, edata[1], cnts[1], params)
    x2, i2 = _pool_sc(h, hexf[1], NP_[2], 64)
    h = _gmm_sc(x2, 2, 2, edata[2], cnts[2], params)
    x3, i3 = _pool_sc(h, hexf[2], NP_[3], 128)
    h = _gmm_sc(x3, 3, 3, edata[3], cnts[3], params)
    x4, i4 = _pool_sc(h, hexf[3], NP_[4], 256)
    h = _gmm_sc(x4, 4, 4, edata[4], cnts[4], params)

    yt = _unpool_sc(*unpool_in(h, i4, 4), NP_[3])
    h = _gmm_sc(x3, 3, 5, edata[3], cnts[3], params, yt=yt)
    yt = _unpool_sc(*unpool_in(h, i3, 3), NP_[2])
    h = _gmm_sc(x2, 2, 6, edata[2], cnts[2], params, yt=yt)
    yt = _unpool_sc(*unpool_in(h, i2, 2), NP_[1])
    h = _gmm_sc(x1, 1, 7, edata[1], cnts[1], params, yt=yt)
    yt = _unpool_sc(*unpool_in(h, i1, 1), NP_[0])
    out = _gmm_sc(x0, 0, 8, edata[0], cnts[0], params, act="softmax", ncls=21,
                  yt=yt)
    return out[:NLV[0]]
